# baseline (speedup 1.0000x reference)
import functools

import jax
import jax.numpy as jnp
from jax import lax
from jax.experimental import pallas as pl
from jax.experimental.pallas import tpu as pltpu
from jax.experimental.pallas import tpu_sc as plsc

_B = 16384
_D = 64
_NW = 32
_BPW = _B // _NW
_ICH = 128
_NCH = _BPW // _ICH

@functools.cache
def _make_gather2():
    mesh = plsc.VectorSubcoreMesh(core_axis_name="c", subcore_axis_name="s")

    @functools.partial(
        pl.kernel,
        mesh=mesh,
        out_type=(
            jax.ShapeDtypeStruct((_B, _D), jnp.float32),
            jax.ShapeDtypeStruct((_B, _D), jnp.float32),
        ),
        scratch_types=[
            pltpu.VMEM((_NCH, _ICH), jnp.int32),
            pltpu.VMEM((_NCH, _ICH), jnp.int32),
            pltpu.VMEM((_BPW, _D), jnp.float32),
            pltpu.VMEM((_BPW, _D), jnp.float32),
            pltpu.SemaphoreType.DMA,
        ],
        compiler_params=pltpu.CompilerParams(use_tc_tiling_on_sc=False),
    )
    def gather2(utab, itab, uids, iids, u_out, i_out, uidx, iidx, urows,
                irows, sem):
        wid = lax.axis_index("s") * 2 + lax.axis_index("c")
        rbase = wid * _NCH
        pltpu.sync_copy(uids.at[pl.ds(rbase, _NCH)], uidx)
        pltpu.sync_copy(iids.at[pl.ds(rbase, _NCH)], iidx)
        copies = []
        for j in range(_NCH):
            copies.append(
                pltpu.async_copy(utab.at[uidx.at[j]],
                                 urows.at[pl.ds(j * _ICH, _ICH)], sem))
            copies.append(
                pltpu.async_copy(itab.at[iidx.at[j]],
                                 irows.at[pl.ds(j * _ICH, _ICH)], sem))
        for c in copies:
            c.wait()
        base = wid * _BPW
        pltpu.sync_copy(urows, u_out.at[pl.ds(base, _BPW)])
        pltpu.sync_copy(irows, i_out.at[pl.ds(base, _BPW)])

    return gather2


_CHUNK = 2048


def _mlp_body(u_ref, i_ref, f_ref, w1_ref, b1_ref, w2_ref, b2_ref, o_ref):
    w1 = w1_ref[...]
    h = jnp.dot(u_ref[...], w1[0:_D, :], preferred_element_type=jnp.float32)
    h = h + jnp.dot(i_ref[...], w1[_D:2 * _D, :],
                    preferred_element_type=jnp.float32)
    h = h + jnp.dot(f_ref[...], w1[2 * _D:3 * _D, :],
                    preferred_element_type=jnp.float32)
    h = jnp.maximum(h + b1_ref[...], 0.0)
    z = jnp.dot(h, w2_ref[...], preferred_element_type=jnp.float32)
    z = z + b2_ref[...]
    o_ref[...] = 1.0 / (1.0 + jnp.exp(-z))


def _mlp(u, i, f, w1, b1, w2, b2):
    grid = (_B // _CHUNK,)
    return pl.pallas_call(
        _mlp_body,
        grid=grid,
        in_specs=[
            pl.BlockSpec((_CHUNK, _D), lambda g: (g, 0)),
            pl.BlockSpec((_CHUNK, _D), lambda g: (g, 0)),
            pl.BlockSpec((_CHUNK, _D), lambda g: (g, 0)),
            pl.BlockSpec((3 * _D, 128), lambda g: (0, 0)),
            pl.BlockSpec((1, 128), lambda g: (0, 0)),
            pl.BlockSpec((128, 1), lambda g: (0, 0)),
            pl.BlockSpec((1, 1), lambda g: (0, 0)),
        ],
        out_specs=pl.BlockSpec((_CHUNK, 1), lambda g: (g, 0)),
        out_shape=jax.ShapeDtypeStruct((_B, 1), jnp.float32),
    )(u, i, f, w1, b1, w2, b2)


def kernel(user_ids, item_ids, item_features, user_table, item_table,
           W1, b1, W2, b2):
    uids = user_ids.astype(jnp.int32).reshape(_B // _ICH, _ICH)
    iids = item_ids.astype(jnp.int32).reshape(_B // _ICH, _ICH)
    u, i = _make_gather2()(user_table, item_table, uids, iids)
    out = _mlp(u, i, item_features, W1, b1.reshape(1, 128), W2,
               b2.reshape(1, 1))
    return out.reshape(_B)
